# baseline (device time: 46348 ns/iter reference)
import os

import jax
import jax.numpy as jnp
from jax import lax
from jax.experimental import pallas as pl
from jax.experimental.pallas import tpu as pltpu

_ABLATE_NOCOMM = os.environ.get("KERNEL_ABLATE", "") == "nocomm"

N_DEV = 32
B, Sq, Hq, Hkv, Dh = 2, 256, 8, 2, 64
Dm, Dq = 768, 512
BH = B * Hq
ROWS = 72
HLF = Sq // 2
GQA = Hq // Hkv


def kernel(x, Wq, Wo, K_ext, V_ext):
    def body(x_ref, wq_ref, wo_ref, k_ref, v_ref, out_ref,
             sbuf, rbuf, cbuf, rs_send, rs_recv, ag_send, ag_recv):
        my = lax.axis_index("i")

        if not _ABLATE_NOCOMM:
            barrier = pltpu.get_barrier_semaphore()
            for t in range(1, N_DEV):
                pl.semaphore_signal(
                    barrier, inc=1,
                    device_id=(my ^ t,),
                    device_id_type=pl.DeviceIdType.MESH,
                )
            pl.semaphore_wait(barrier, N_DEV - 1)

        for b in range(B):
            qb = jnp.dot(x_ref[b], wq_ref[...],
                         preferred_element_type=jnp.float32)
            for h in range(Hq):
                kh = h // GQA
                q = qb[:, h * Dh:(h + 1) * Dh]
                kk = k_ref[b, :, kh, :]
                vv = v_ref[b, :, kh, :]
                sT = lax.dot_general(
                    kk, q, (((1,), (1,)), ((), ())),
                    preferred_element_type=jnp.float32) * 0.125
                m = jnp.max(sT, axis=0, keepdims=True)
                p = jnp.exp(sT - m)
                l = jnp.sum(p, axis=0, keepdims=True)
                oT = lax.dot_general(
                    vv, p, (((0,), (0,)), ((), ())),
                    preferred_element_type=jnp.float32)
                i = b * Hq + h
                oT16 = oT.astype(jnp.bfloat16)
                m16 = m.astype(jnp.bfloat16)
                l16 = l.astype(jnp.bfloat16)
                for j in range(2):
                    pc = 2 * i + j
                    cols = slice(j * HLF, (j + 1) * HLF)
                    sbuf[pc, 0:Dh, :] = oT16[:, cols]
                    sbuf[pc, Dh:Dh + 1, :] = m16[:, cols]
                    sbuf[pc, Dh + 1:Dh + 2, :] = l16[:, cols]

        def flash_combine(A, R):
            a_m = A[:, Dh:Dh + 1, :]
            r_m = R[:, Dh:Dh + 1, :]
            m_new = jnp.maximum(a_m, r_m)
            aa = jnp.exp(a_m - m_new)
            ar = jnp.exp(r_m - m_new)
            o_new = A[:, 0:Dh, :] * aa + R[:, 0:Dh, :] * ar
            l_new = A[:, Dh + 1:Dh + 2, :] * aa + R[:, Dh + 1:Dh + 2, :] * ar
            return o_new, m_new, l_new

        rs_descs = []
        if not _ABLATE_NOCOMM:
            for t in range(1, N_DEV):
                rdma = pltpu.make_async_remote_copy(
                    src_ref=sbuf.at[pl.ds(my ^ t, 1)],
                    dst_ref=rbuf.at[pl.ds(t - 1, 1)],
                    send_sem=rs_send.at[t - 1],
                    recv_sem=rs_recv.at[t - 1],
                    device_id=(my ^ t,),
                    device_id_type=pl.DeviceIdType.MESH,
                )
                rdma.start()
                rs_descs.append(rdma)
            for rdma in rs_descs:
                rdma.wait_recv()

        rbuf[pl.ds(N_DEV - 1, 1)] = sbuf[pl.ds(my, 1)]
        o_new, m_new, l_new = flash_combine(
            rbuf[pl.ds(0, 16)].astype(jnp.float32),
            rbuf[pl.ds(16, 16)].astype(jnp.float32))
        cbuf[:, 0:Dh, :] = o_new
        cbuf[:, Dh:Dh + 1, :] = m_new
        cbuf[:, Dh + 1:Dh + 2, :] = l_new
        for L in (8, 4, 2, 1):
            o_new, m_new, l_new = flash_combine(
                cbuf[pl.ds(0, L)], cbuf[pl.ds(L, L)])
            cbuf[pl.ds(0, L), 0:Dh, :] = o_new
            cbuf[pl.ds(0, L), Dh:Dh + 1, :] = m_new
            cbuf[pl.ds(0, L), Dh + 1:Dh + 2, :] = l_new
        sbuf[pl.ds(my, 1), 0:Dh + 2, :] = (
            cbuf[pl.ds(0, 1), 0:Dh + 2, :].astype(jnp.bfloat16))

        ag_descs = []
        if not _ABLATE_NOCOMM:
            for t in range(1, N_DEV):
                rdma = pltpu.make_async_remote_copy(
                    src_ref=sbuf.at[pl.ds(my, 1)],
                    dst_ref=sbuf.at[pl.ds(my, 1)],
                    send_sem=ag_send.at[t - 1],
                    recv_sem=ag_recv.at[t - 1],
                    device_id=(my ^ t,),
                    device_id_type=pl.DeviceIdType.MESH,
                )
                rdma.start()
                ag_descs.append(rdma)
            for rdma in ag_descs:
                rdma.wait_recv()
            for rdma in rs_descs + ag_descs:
                rdma.wait_send()

        for b in range(B):
            acc = jnp.zeros((Sq, Dm), dtype=jnp.float32)
            for h in range(Hq):
                i = b * Hq + h
                o = jnp.concatenate(
                    [sbuf[2 * i, 0:Dh, :], sbuf[2 * i + 1, 0:Dh, :]],
                    axis=1).astype(jnp.float32)
                l = jnp.concatenate(
                    [sbuf[2 * i, Dh + 1:Dh + 2, :],
                     sbuf[2 * i + 1, Dh + 1:Dh + 2, :]],
                    axis=1).astype(jnp.float32)
                acc = acc + lax.dot_general(
                    o / l, wo_ref[h * Dh:(h + 1) * Dh, :],
                    (((0,), (0,)), ((), ())),
                    preferred_element_type=jnp.float32)
            out_ref[b, :, :] = acc

    return pl.pallas_call(
        body,
        out_shape=jax.ShapeDtypeStruct((B, Sq, Dm), jnp.float32),
        in_specs=[pl.BlockSpec(memory_space=pltpu.VMEM)] * 5,
        out_specs=pl.BlockSpec(memory_space=pltpu.VMEM),
        scratch_shapes=[
            pltpu.VMEM((2 * BH, ROWS, HLF), jnp.bfloat16),
            pltpu.VMEM((N_DEV, ROWS, HLF), jnp.bfloat16),
            pltpu.VMEM((16, ROWS, HLF), jnp.float32),
            pltpu.SemaphoreType.DMA((N_DEV - 1,)),
            pltpu.SemaphoreType.DMA((N_DEV - 1,)),
            pltpu.SemaphoreType.DMA((N_DEV - 1,)),
            pltpu.SemaphoreType.DMA((N_DEV - 1,)),
        ],
        compiler_params=(None if _ABLATE_NOCOMM
                         else pltpu.CompilerParams(collective_id=0)),
    )(x, Wq, Wo, K_ext, V_ext)
